# Initial kernel scaffold; baseline (speedup 1.0000x reference)
import functools
from math import sqrt

import jax
import jax.numpy as jnp
from jax import lax
from jax.experimental import pallas as pl
from jax.experimental.pallas import tpu as pltpu
from jax.experimental.pallas import tpu_sc as plsc

N = 10000
E = 320000
D = 128
F = 20000
H = 4
HD = 32
NP = 10240
DUMMY = 10016
R = 144
NC = 2
NS = 16
NW = NC * NS
EPW = 10240
E_PAD = NW * EPW
CH = 128
NCH = EPW // CH
FP = 20480
FPW = FP // NW
BR = 2048
NB = NP // BR
HP = 384

_mesh = plsc.VectorSubcoreMesh(core_axis_name="c", subcore_axis_name="s",
                               num_cores=NC, num_subcores=NS)


def _full(v):
    return jnp.full((16,), v, jnp.int32)


@functools.partial(
    pl.kernel,
    out_type=jax.ShapeDtypeStruct((NC, NP, R), jnp.float32),
    mesh=_mesh,
    scratch_types=[
        pltpu.VMEM((CH,), jnp.int32),
        pltpu.VMEM((CH,), jnp.int32),
        pltpu.VMEM((CH, R), jnp.float32),
        pltpu.VMEM((CH, 16), jnp.float32),
        pltpu.VMEM((16,), jnp.float32),
        pltpu.VMEM_SHARED((NP, R), jnp.float32),
        pltpu.SemaphoreType.DMA,
        pltpu.SemaphoreType.DMA,
    ],
)
def _sc_hgat(tsrc, adst, src, dst, btile, out, idx_s, idx_d, bufS, bufD,
             bvec, accum, sem1, sem2):
    c = lax.axis_index("c")
    s = lax.axis_index("s")
    w = c * NS + s

    pltpu.sync_copy(btile, bvec)

    z16 = jnp.zeros((16,), jnp.float32)

    @pl.loop(0, CH)
    def _(r):
        for jj in range(R // 16):
            bufS[r, pl.ds(jj * 16, 16)] = z16

    @pl.loop(0, NP // NS // CH)
    def _(zb):
        pltpu.sync_copy(bufS, accum.at[pl.ds(s * (NP // NS) + zb * CH, CH)])

    plsc.subcore_barrier()

    la = lax.iota(jnp.int32, (16,))
    e_off = lax.shift_right_logical(la, 2)
    hh = lax.bitwise_and(la, 3)
    bv = bvec[...]

    @pl.loop(0, NCH)
    def _(ch):
        base = w * EPW + ch * CH
        pltpu.sync_copy(src.at[pl.ds(base, CH)], idx_s)
        pltpu.sync_copy(dst.at[pl.ds(base, CH)], idx_d)
        cp1 = pltpu.async_copy(tsrc.at[idx_s], bufS, sem1)
        cp2 = pltpu.async_copy(adst.at[idx_d], bufD, sem2)
        cp1.wait()
        cp2.wait()

        @pl.loop(0, CH // 4)
        def _(i):
            rows = e_off + i * 4
            colS = hh + 128
            aS = plsc.load_gather(bufS, [rows, colS])
            aD = plsc.load_gather(bufD, [rows, hh])
            zv = aS + aD + bv
            lk = jnp.where(zv >= 0.0, zv, zv * 0.2)
            plsc.store_scatter(bufS, [rows, colS], jnp.exp(lk))

        @pl.loop(0, CH)
        def _(e):
            re = _full(e)
            for h in range(H):
                sp = plsc.load_gather(bufS, [re, _full(128 + h)])
                for q in range(2):
                    cc = h * 32 + q * 16
                    bufS[e, pl.ds(cc, 16)] = bufS[e, pl.ds(cc, 16)] * sp

        pltpu.sync_copy(bufS, accum.at[idx_d], add=True)

    plsc.subcore_barrier()
    pltpu.sync_copy(accum.at[pl.ds(s * (NP // NS), NP // NS)],
                    out.at[c, pl.ds(s * (NP // NS), NP // NS)])


@functools.partial(
    pl.kernel,
    out_type=jax.ShapeDtypeStruct((NC, NP, R), jnp.float32),
    mesh=_mesh,
    scratch_types=[
        pltpu.VMEM((CH,), jnp.int32),
        pltpu.VMEM((CH,), jnp.int32),
        pltpu.VMEM((CH, D), jnp.float32),
        pltpu.VMEM((CH, 2 * D), jnp.float32),
        pltpu.VMEM((CH, HD), jnp.float32),
        pltpu.VMEM((CH, R), jnp.float32),
        pltpu.VMEM_SHARED((NP, R), jnp.float32),
        pltpu.SemaphoreType.DMA,
        pltpu.SemaphoreType.DMA,
    ],
)
def _sc_mha(qt, kvt, src, dst, attr, out, idx_s, idx_d, bufQ, bufKV, bufA,
            bufU, accum, sem1, sem2):
    c = lax.axis_index("c")
    s = lax.axis_index("s")
    w = c * NS + s

    z16 = jnp.zeros((16,), jnp.float32)

    @pl.loop(0, CH)
    def _(r):
        for jj in range(R // 16):
            bufU[r, pl.ds(jj * 16, 16)] = z16

    @pl.loop(0, NP // NS // CH)
    def _(zb):
        pltpu.sync_copy(bufU, accum.at[pl.ds(s * (NP // NS) + zb * CH, CH)])

    plsc.subcore_barrier()

    la = lax.iota(jnp.int32, (16,))

    @pl.loop(0, NCH)
    def _(ch):
        base = w * EPW + ch * CH
        pltpu.sync_copy(src.at[pl.ds(base, CH)], idx_s)
        pltpu.sync_copy(dst.at[pl.ds(base, CH)], idx_d)
        cp1 = pltpu.async_copy(qt.at[idx_d], bufQ, sem1)
        cp2 = pltpu.async_copy(kvt.at[idx_s], bufKV, sem2)
        pltpu.sync_copy(attr.at[pl.ds(base, CH)], bufA)
        cp1.wait()
        cp2.wait()

        @pl.loop(0, CH // 16)
        def _(g):
            er = la + g * 16
            accs = [jnp.zeros((16,), jnp.float32) for _ in range(H)]
            for d in range(HD):
                ac = plsc.load_gather(bufA, [er, _full(d)])
                for h in range(H):
                    qc = plsc.load_gather(bufQ, [er, _full(h * 32 + d)])
                    kc = plsc.load_gather(bufKV, [er, _full(h * 32 + d)])
                    accs[h] = accs[h] + qc * kc * ac
            for h in range(H):
                plsc.store_scatter(bufU, [er, _full(128 + h)],
                                   jnp.exp(accs[h]))

        @pl.loop(0, CH)
        def _(e):
            re = _full(e)
            for h in range(H):
                sp = plsc.load_gather(bufU, [re, _full(128 + h)])
                for q in range(2):
                    cc = h * 32 + q * 16
                    bufU[e, pl.ds(cc, 16)] = bufKV[e, pl.ds(128 + cc, 16)] * sp

        pltpu.sync_copy(bufU, accum.at[idx_d], add=True)

    plsc.subcore_barrier()
    pltpu.sync_copy(accum.at[pl.ds(s * (NP // NS), NP // NS)],
                    out.at[c, pl.ds(s * (NP // NS), NP // NS)])


@functools.partial(
    pl.kernel,
    out_type=jax.ShapeDtypeStruct((FP, D), jnp.float32),
    mesh=_mesh,
    scratch_types=[
        pltpu.VMEM((CH,), jnp.int32),
        pltpu.VMEM((CH, D), jnp.float32),
        pltpu.VMEM((CH, D), jnp.float32),
        pltpu.SemaphoreType.DMA,
    ],
)
def _sc_fringe(proj, fidx, maps, out, idxb, bufP, bufM, sem):
    c = lax.axis_index("c")
    s = lax.axis_index("s")
    w = c * NS + s

    @pl.loop(0, FPW // CH)
    def _(ch):
        base = w * FPW + ch * CH
        pltpu.sync_copy(fidx.at[pl.ds(base, CH)], idxb)
        cp = pltpu.async_copy(proj.at[idxb], bufP, sem)
        pltpu.sync_copy(maps.at[pl.ds(base, CH)], bufM)
        cp.wait()

        @pl.loop(0, CH)
        def _(r):
            for jj in range(D // 16):
                bufP[r, pl.ds(jj * 16, 16)] = (
                    bufP[r, pl.ds(jj * 16, 16)] * bufM[r, pl.ds(jj * 16, 16)])

        pltpu.sync_copy(bufP, out.at[pl.ds(base, CH)])


def _mm(a, b):
    return jnp.dot(a, b, preferred_element_type=jnp.float32,
                   precision=lax.Precision.HIGHEST)


def _rms(x, w):
    return x * lax.rsqrt(jnp.mean(x * x, axis=-1, keepdims=True) + 1e-6) * w


def _bcast_heads(a4, nrows):
    return jnp.concatenate(
        [jnp.broadcast_to(a4[:, h:h + 1], (nrows, HD)) for h in range(H)],
        axis=1)


def _tc_prep_body(root, ctx, wc2x, bc2x, wx2c, bx2c, was, wad, tsrc, adst):
    cx = ctx[...]
    vals = _mm(cx, wc2x[...]) + bc2x[...]
    a_s = _mm(cx, was[...])
    tsrc[...] = jnp.concatenate([vals, a_s], axis=1)
    pre = _mm(root[...], wx2c[...]) + bx2c[...]
    adst[...] = _mm(pre, wad[...])


def _tc_mid_body(acc, root, adst, b16, wn, wq, bq, wk, bk, wv, bv,
                 rf1o, qo, kvo):
    a = acc[...]
    ac = a[0] + a[1]
    rt = root[...]
    zs = adst[...] + b16[...]
    ex16 = jnp.exp(jnp.where(zs >= 0.0, zs, zs * 0.2))
    ex4 = ex16[:, :H]
    numer = ac[:, :D] + _bcast_heads(ex4, BR) * rt
    denom = ac[:, D:D + H] + ex4
    hout = numer / (_bcast_heads(denom, BR) + 1e-16)
    rf1 = _rms(rt + hout, wn[...])
    rf1o[...] = rf1
    qo[...] = (_mm(rf1, wq[...]) + bq[...]) * (1.0 / sqrt(HD))
    kvo[...] = jnp.concatenate(
        [_mm(rf1, wk[...]) + bk[...], _mm(rf1, wv[...]) + bv[...]], axis=1)


def _tc_tail_body(acc, rf1, wn2, wn3, w1, w3, w2, wr2f, br2f, rf3o, projo):
    a = acc[...]
    ac = a[0] + a[1]
    numer = ac[:, :D]
    denom = ac[:, D:D + H]
    mo = numer / (_bcast_heads(denom, BR) + 1e-16)
    rf2 = _rms(rf1[...] + mo, wn2[...])
    h1 = _mm(rf2, w1[...])
    hs = h1 * jax.nn.sigmoid(h1) * _mm(rf2, w3[...])
    rf3 = _rms(rf2 + _mm(hs, w2[...]), wn3[...])
    rf3o[...] = rf3
    projo[...] = _mm(rf3, wr2f[...]) + br2f[...]


def _row_spec(cols):
    return pl.BlockSpec((BR, cols), lambda i: (i, 0))


def _w_spec(shape):
    nd = len(shape)
    return pl.BlockSpec(shape, lambda i: (0,) * nd)


def _acc_spec():
    return pl.BlockSpec((NC, BR, R), lambda i: (0, i, 0))


def _tc_prep(root_p, ctx_p, p):
    was = jnp.pad(p["W_attn"][:D], ((0, 0), (0, 16 - H)))
    wad = jnp.pad(p["W_attn"][D:], ((0, 0), (0, 16 - H)))
    return pl.pallas_call(
        _tc_prep_body,
        grid=(NB,),
        in_specs=[_row_spec(D), _row_spec(D), _w_spec((D, D)), _w_spec((1, D)),
                  _w_spec((D, D)), _w_spec((1, D)), _w_spec((D, 16)),
                  _w_spec((D, 16))],
        out_specs=[_row_spec(R), _row_spec(16)],
        out_shape=[jax.ShapeDtypeStruct((NP, R), jnp.float32),
                   jax.ShapeDtypeStruct((NP, 16), jnp.float32)],
    )(root_p, ctx_p, p["W_ctx_to_x"], p["b_ctx_to_x"].reshape(1, D),
      p["W_x_to_ctx"], p["b_x_to_ctx"].reshape(1, D), was, wad)


def _tc_mid(acc, root_p, adst, b_attn, node_w, mha_p):
    wqkv = mha_p["W_qkv"].reshape(D, H, HD, 3)
    bqkv = mha_p["b_qkv"].reshape(H, HD, 3)
    wq = wqkv[..., 0].reshape(D, D)
    wk = wqkv[..., 1].reshape(D, D)
    wv = wqkv[..., 2].reshape(D, D)
    bq = bqkv[..., 0].reshape(1, D)
    bk = bqkv[..., 1].reshape(1, D)
    bv = bqkv[..., 2].reshape(1, D)
    b16 = jnp.pad(b_attn, (0, 16 - H)).reshape(1, 16)
    return pl.pallas_call(
        _tc_mid_body,
        grid=(NB,),
        in_specs=[_acc_spec(), _row_spec(D), _row_spec(16), _w_spec((1, 16)),
                  _w_spec((1, D)), _w_spec((D, D)), _w_spec((1, D)),
                  _w_spec((D, D)), _w_spec((1, D)), _w_spec((D, D)),
                  _w_spec((1, D))],
        out_specs=[_row_spec(D), _row_spec(D), _row_spec(2 * D)],
        out_shape=[jax.ShapeDtypeStruct((NP, D), jnp.float32),
                   jax.ShapeDtypeStruct((NP, D), jnp.float32),
                   jax.ShapeDtypeStruct((NP, 2 * D), jnp.float32)],
    )(acc, root_p, adst, b16, node_w.reshape(1, D), wq, bq, wk, bk, wv, bv)


def _tc_tail(acc2, rf1, root_w, ffn_w, ffn_p, r2f_p):
    w1 = jnp.pad(ffn_p["W1"], ((0, 0), (0, HP - ffn_p["W1"].shape[1])))
    w3 = jnp.pad(ffn_p["W3"], ((0, 0), (0, HP - ffn_p["W3"].shape[1])))
    w2 = jnp.pad(ffn_p["W2"], ((0, HP - ffn_p["W2"].shape[0]), (0, 0)))
    return pl.pallas_call(
        _tc_tail_body,
        grid=(NB,),
        in_specs=[_acc_spec(), _row_spec(D), _w_spec((1, D)), _w_spec((1, D)),
                  _w_spec((D, HP)), _w_spec((D, HP)), _w_spec((HP, D)),
                  _w_spec((D, D)), _w_spec((1, D))],
        out_specs=[_row_spec(D), _row_spec(D)],
        out_shape=[jax.ShapeDtypeStruct((NP, D), jnp.float32),
                   jax.ShapeDtypeStruct((NP, D), jnp.float32)],
    )(acc2, rf1, root_w.reshape(1, D), ffn_w.reshape(1, D), w1, w3, w2,
      r2f_p["W"], r2f_p["b"].reshape(1, D))


@jax.jit
def _run(root_features, feedback_features, feedback_index, fringe_maps,
         root_to_fringe_index, root_edge_index, root_edge_attr, params):
    root_p = jnp.pad(root_features, ((0, NP - N), (0, 0)))
    ctx_p = jnp.pad(feedback_features, ((0, NP - N), (0, 0)))

    def pad_edges(ei):
        srcp = jnp.concatenate(
            [ei[0].astype(jnp.int32), jnp.zeros((E_PAD - E,), jnp.int32)])
        dstp = jnp.concatenate(
            [ei[1].astype(jnp.int32), jnp.full((E_PAD - E,), DUMMY, jnp.int32)])
        return srcp, dstp

    src1, dst1 = pad_edges(feedback_index)
    src2, dst2 = pad_edges(root_edge_index)
    attr_p = jnp.pad(root_edge_attr, ((0, E_PAD - E), (0, 0)))
    fidx_p = jnp.concatenate(
        [root_to_fringe_index.astype(jnp.int32),
         jnp.zeros((FP - F,), jnp.int32)])
    maps_p = jnp.pad(fringe_maps, ((0, FP - F), (0, 0)))

    hp = params["hgat"]
    btile = jnp.tile(hp["b_attn"], H).astype(jnp.float32)

    tsrc, adst = _tc_prep(root_p, ctx_p, hp)
    acc1 = _sc_hgat(tsrc, adst, src1, dst1, btile)
    rf1, qt, kvt = _tc_mid(acc1, root_p, adst, hp["b_attn"],
                           params["node_fb_norm_w"], params["mha"])
    acc2 = _sc_mha(qt, kvt, src2, dst2, attr_p)
    rf3, proj = _tc_tail(acc2, rf1, params["root_fb_norm_w"],
                         params["ffn_norm_w"], params["ffn"],
                         params["root_to_fringe"])
    fringe = _sc_fringe(proj, fidx_p, maps_p)
    return rf3[:N], fringe[:F]


def kernel(root_features, feedback_features, feedback_index, fringe_maps,
           root_to_fringe_index, root_edge_index, root_edge_attr, params):
    return _run(root_features, feedback_features, feedback_index, fringe_maps,
                root_to_fringe_index, root_edge_index, root_edge_attr, params)


# speedup vs baseline: 19.9794x; 19.9794x over previous
import functools
from math import sqrt

import jax
import jax.numpy as jnp
from jax import lax
from jax.experimental import pallas as pl
from jax.experimental.pallas import tpu as pltpu
from jax.experimental.pallas import tpu_sc as plsc

N = 10000
E = 320000
D = 128
F = 20000
H = 4
HD = 32
NP = 10240
DUMMY = 10016
R = 144
NC = 2
NS = 16
NW = NC * NS
EPW = 10240
E_PAD = NW * EPW
CH = 128
NCH = EPW // CH
CHM = 64
NCHM = EPW // CHM
FP = 20480
FPW = FP // NW
BR = 2048
NB = NP // BR
HP = 384

_mesh = plsc.VectorSubcoreMesh(core_axis_name="c", subcore_axis_name="s",
                               num_cores=NC, num_subcores=NS)


def _full(v):
    return jnp.full((16,), v, jnp.int32)


@functools.partial(
    pl.kernel,
    out_type=jax.ShapeDtypeStruct((NC, NP, R), jnp.float32),
    mesh=_mesh,
    compiler_params=pltpu.CompilerParams(use_tc_tiling_on_sc=False, needs_layout_passes=False),
    scratch_types=[
        pltpu.VMEM((CH,), jnp.int32),
        pltpu.VMEM((CH,), jnp.int32),
        pltpu.VMEM((CH, R), jnp.float32),
        pltpu.VMEM((CH, 16), jnp.float32),
        pltpu.VMEM((16,), jnp.float32),
        pltpu.VMEM_SHARED((NP, R), jnp.float32),
        pltpu.SemaphoreType.DMA,
        pltpu.SemaphoreType.DMA,
    ],
)
def _sc_hgat(tsrc, adst, src, dst, btile, out, idx_s, idx_d, bufS, bufD,
             bvec, accum, sem1, sem2):
    c = lax.axis_index("c")
    s = lax.axis_index("s")
    w = c * NS + s

    pltpu.sync_copy(btile, bvec)

    z16 = jnp.zeros((16,), jnp.float32)

    @pl.loop(0, CH)
    def _(r):
        for jj in range(R // 16):
            bufS[r, pl.ds(jj * 16, 16)] = z16

    @pl.loop(0, NP // NS // CH)
    def _(zb):
        pltpu.sync_copy(bufS, accum.at[pl.ds(s * (NP // NS) + zb * CH, CH)])

    plsc.subcore_barrier()

    la = lax.iota(jnp.int32, 16)
    e_off = lax.shift_right_logical(la, 2)
    hh = lax.bitwise_and(la, 3)
    bv = bvec[...]

    @pl.loop(0, NCH)
    def _(ch):
        base = w * EPW + ch * CH
        pltpu.sync_copy(src.at[pl.ds(base, CH)], idx_s)
        pltpu.sync_copy(dst.at[pl.ds(base, CH)], idx_d)
        cp1 = pltpu.async_copy(tsrc.at[idx_s], bufS, sem1)
        cp2 = pltpu.async_copy(adst.at[idx_d], bufD, sem2)
        cp1.wait()
        cp2.wait()

        @pl.loop(0, CH // 4)
        def _(i):
            rows = e_off + i * 4
            colS = hh + 128
            aS = plsc.load_gather(bufS, [rows, colS])
            aD = plsc.load_gather(bufD, [rows, hh])
            zv = aS + aD + bv
            lk = jnp.where(zv >= 0.0, zv, zv * 0.2)
            plsc.store_scatter(bufS, [rows, colS], jnp.exp(lk))

        @pl.loop(0, CH)
        def _(e):
            re = _full(e)
            for h in range(H):
                sp = plsc.load_gather(bufS, [re, _full(128 + h)])
                for q in range(2):
                    cc = h * 32 + q * 16
                    bufS[e, pl.ds(cc, 16)] = bufS[e, pl.ds(cc, 16)] * sp

        pltpu.sync_copy(bufS, accum.at[idx_d], add=True)

    plsc.subcore_barrier()
    pltpu.sync_copy(accum.at[pl.ds(s * (NP // NS), NP // NS)],
                    out.at[c, pl.ds(s * (NP // NS), NP // NS)])


@functools.partial(
    pl.kernel,
    out_type=jax.ShapeDtypeStruct((NC, NP, R), jnp.float32),
    mesh=_mesh,
    compiler_params=pltpu.CompilerParams(use_tc_tiling_on_sc=False, needs_layout_passes=False),
    scratch_types=[
        pltpu.VMEM((CHM,), jnp.int32),
        pltpu.VMEM((CHM,), jnp.int32),
        pltpu.VMEM((CHM, D), jnp.float32),
        pltpu.VMEM((CHM, 2 * D), jnp.float32),
        pltpu.VMEM((CHM, HD), jnp.float32),
        pltpu.VMEM((CHM, R), jnp.float32),
        pltpu.VMEM_SHARED((NP, R), jnp.float32),
        pltpu.SemaphoreType.DMA,
        pltpu.SemaphoreType.DMA,
    ],
)
def _sc_mha(qt, kvt, src, dst, attr, out, idx_s, idx_d, bufQ, bufKV, bufA,
            bufU, accum, sem1, sem2):
    c = lax.axis_index("c")
    s = lax.axis_index("s")
    w = c * NS + s

    z16 = jnp.zeros((16,), jnp.float32)

    @pl.loop(0, CHM)
    def _(r):
        for jj in range(R // 16):
            bufU[r, pl.ds(jj * 16, 16)] = z16

    @pl.loop(0, NP // NS // CHM)
    def _(zb):
        pltpu.sync_copy(bufU, accum.at[pl.ds(s * (NP // NS) + zb * CHM, CHM)])

    plsc.subcore_barrier()

    la = lax.iota(jnp.int32, 16)

    @pl.loop(0, NCHM)
    def _(ch):
        base = w * EPW + ch * CHM
        pltpu.sync_copy(src.at[pl.ds(base, CHM)], idx_s)
        pltpu.sync_copy(dst.at[pl.ds(base, CHM)], idx_d)
        cp1 = pltpu.async_copy(qt.at[idx_d], bufQ, sem1)
        cp2 = pltpu.async_copy(kvt.at[idx_s], bufKV, sem2)
        pltpu.sync_copy(attr.at[pl.ds(base, CHM)], bufA)
        cp1.wait()
        cp2.wait()

        @pl.loop(0, CHM // 16)
        def _(g):
            er = la + g * 16
            accs = [jnp.zeros((16,), jnp.float32) for _ in range(H)]
            for d in range(HD):
                ac = plsc.load_gather(bufA, [er, _full(d)])
                for h in range(H):
                    qc = plsc.load_gather(bufQ, [er, _full(h * 32 + d)])
                    kc = plsc.load_gather(bufKV, [er, _full(h * 32 + d)])
                    accs[h] = accs[h] + qc * kc * ac
            for h in range(H):
                plsc.store_scatter(bufU, [er, _full(128 + h)],
                                   jnp.exp(accs[h]))

        @pl.loop(0, CHM)
        def _(e):
            re = _full(e)
            for h in range(H):
                sp = plsc.load_gather(bufU, [re, _full(128 + h)])
                for q in range(2):
                    cc = h * 32 + q * 16
                    bufU[e, pl.ds(cc, 16)] = bufKV[e, pl.ds(128 + cc, 16)] * sp

        pltpu.sync_copy(bufU, accum.at[idx_d], add=True)

    plsc.subcore_barrier()
    pltpu.sync_copy(accum.at[pl.ds(s * (NP // NS), NP // NS)],
                    out.at[c, pl.ds(s * (NP // NS), NP // NS)])


@functools.partial(
    pl.kernel,
    out_type=jax.ShapeDtypeStruct((FP, D), jnp.float32),
    mesh=_mesh,
    compiler_params=pltpu.CompilerParams(use_tc_tiling_on_sc=False, needs_layout_passes=False),
    scratch_types=[
        pltpu.VMEM((CH,), jnp.int32),
        pltpu.VMEM((CH, D), jnp.float32),
        pltpu.VMEM((CH, D), jnp.float32),
        pltpu.SemaphoreType.DMA,
    ],
)
def _sc_fringe(proj, fidx, maps, out, idxb, bufP, bufM, sem):
    c = lax.axis_index("c")
    s = lax.axis_index("s")
    w = c * NS + s

    @pl.loop(0, FPW // CH)
    def _(ch):
        base = w * FPW + ch * CH
        pltpu.sync_copy(fidx.at[pl.ds(base, CH)], idxb)
        cp = pltpu.async_copy(proj.at[idxb], bufP, sem)
        pltpu.sync_copy(maps.at[pl.ds(base, CH)], bufM)
        cp.wait()

        @pl.loop(0, CH)
        def _(r):
            for jj in range(D // 16):
                bufP[r, pl.ds(jj * 16, 16)] = (
                    bufP[r, pl.ds(jj * 16, 16)] * bufM[r, pl.ds(jj * 16, 16)])

        pltpu.sync_copy(bufP, out.at[pl.ds(base, CH)])


def _mm(a, b):
    return jnp.dot(a, b, preferred_element_type=jnp.float32,
                   precision=lax.Precision.HIGHEST)


def _rms(x, w):
    return x * lax.rsqrt(jnp.mean(x * x, axis=-1, keepdims=True) + 1e-6) * w


def _bcast_heads(a4, nrows):
    return jnp.concatenate(
        [jnp.broadcast_to(a4[:, h:h + 1], (nrows, HD)) for h in range(H)],
        axis=1)


def _tc_prep_body(root, ctx, wc2x, bc2x, wx2c, bx2c, was, wad, tsrc, adst):
    cx = ctx[...]
    vals = _mm(cx, wc2x[...]) + bc2x[...]
    a_s = _mm(cx, was[...])
    tsrc[...] = jnp.concatenate([vals, a_s], axis=1)
    pre = _mm(root[...], wx2c[...]) + bx2c[...]
    adst[...] = _mm(pre, wad[...])


def _tc_mid_body(acc, root, adst, b16, wn, wq, bq, wk, bk, wv, bv,
                 rf1o, qo, kvo):
    a = acc[...]
    ac = a[0] + a[1]
    rt = root[...]
    zs = adst[...] + b16[...]
    ex16 = jnp.exp(jnp.where(zs >= 0.0, zs, zs * 0.2))
    ex4 = ex16[:, :H]
    numer = ac[:, :D] + _bcast_heads(ex4, BR) * rt
    denom = ac[:, D:D + H] + ex4
    hout = numer / (_bcast_heads(denom, BR) + 1e-16)
    rf1 = _rms(rt + hout, wn[...])
    rf1o[...] = rf1
    qo[...] = (_mm(rf1, wq[...]) + bq[...]) * (1.0 / sqrt(HD))
    kvo[...] = jnp.concatenate(
        [_mm(rf1, wk[...]) + bk[...], _mm(rf1, wv[...]) + bv[...]], axis=1)


def _tc_tail_body(acc, rf1, wn2, wn3, w1, w3, w2, wr2f, br2f, rf3o, projo):
    a = acc[...]
    ac = a[0] + a[1]
    numer = ac[:, :D]
    denom = ac[:, D:D + H]
    mo = numer / (_bcast_heads(denom, BR) + 1e-16)
    rf2 = _rms(rf1[...] + mo, wn2[...])
    h1 = _mm(rf2, w1[...])
    hs = h1 * jax.nn.sigmoid(h1) * _mm(rf2, w3[...])
    rf3 = _rms(rf2 + _mm(hs, w2[...]), wn3[...])
    rf3o[...] = rf3
    projo[...] = _mm(rf3, wr2f[...]) + br2f[...]


def _row_spec(cols):
    return pl.BlockSpec((BR, cols), lambda i: (i, 0))


def _w_spec(shape):
    nd = len(shape)
    return pl.BlockSpec(shape, lambda i: (0,) * nd)


def _acc_spec():
    return pl.BlockSpec((NC, BR, R), lambda i: (0, i, 0))


def _tc_prep(root_p, ctx_p, p):
    was = jnp.pad(p["W_attn"][:D], ((0, 0), (0, 16 - H)))
    wad = jnp.pad(p["W_attn"][D:], ((0, 0), (0, 16 - H)))
    return pl.pallas_call(
        _tc_prep_body,
        grid=(NB,),
        in_specs=[_row_spec(D), _row_spec(D), _w_spec((D, D)), _w_spec((1, D)),
                  _w_spec((D, D)), _w_spec((1, D)), _w_spec((D, 16)),
                  _w_spec((D, 16))],
        out_specs=[_row_spec(R), _row_spec(16)],
        out_shape=[jax.ShapeDtypeStruct((NP, R), jnp.float32),
                   jax.ShapeDtypeStruct((NP, 16), jnp.float32)],
    )(root_p, ctx_p, p["W_ctx_to_x"], p["b_ctx_to_x"].reshape(1, D),
      p["W_x_to_ctx"], p["b_x_to_ctx"].reshape(1, D), was, wad)


def _tc_mid(acc, root_p, adst, b_attn, node_w, mha_p):
    wqkv = mha_p["W_qkv"].reshape(D, H, HD, 3)
    bqkv = mha_p["b_qkv"].reshape(H, HD, 3)
    wq = wqkv[..., 0].reshape(D, D)
    wk = wqkv[..., 1].reshape(D, D)
    wv = wqkv[..., 2].reshape(D, D)
    bq = bqkv[..., 0].reshape(1, D)
    bk = bqkv[..., 1].reshape(1, D)
    bv = bqkv[..., 2].reshape(1, D)
    b16 = jnp.pad(b_attn, (0, 16 - H)).reshape(1, 16)
    return pl.pallas_call(
        _tc_mid_body,
        grid=(NB,),
        in_specs=[_acc_spec(), _row_spec(D), _row_spec(16), _w_spec((1, 16)),
                  _w_spec((1, D)), _w_spec((D, D)), _w_spec((1, D)),
                  _w_spec((D, D)), _w_spec((1, D)), _w_spec((D, D)),
                  _w_spec((1, D))],
        out_specs=[_row_spec(D), _row_spec(D), _row_spec(2 * D)],
        out_shape=[jax.ShapeDtypeStruct((NP, D), jnp.float32),
                   jax.ShapeDtypeStruct((NP, D), jnp.float32),
                   jax.ShapeDtypeStruct((NP, 2 * D), jnp.float32)],
    )(acc, root_p, adst, b16, node_w.reshape(1, D), wq, bq, wk, bk, wv, bv)


def _tc_tail(acc2, rf1, root_w, ffn_w, ffn_p, r2f_p):
    w1 = jnp.pad(ffn_p["W1"], ((0, 0), (0, HP - ffn_p["W1"].shape[1])))
    w3 = jnp.pad(ffn_p["W3"], ((0, 0), (0, HP - ffn_p["W3"].shape[1])))
    w2 = jnp.pad(ffn_p["W2"], ((0, HP - ffn_p["W2"].shape[0]), (0, 0)))
    return pl.pallas_call(
        _tc_tail_body,
        grid=(NB,),
        in_specs=[_acc_spec(), _row_spec(D), _w_spec((1, D)), _w_spec((1, D)),
                  _w_spec((D, HP)), _w_spec((D, HP)), _w_spec((HP, D)),
                  _w_spec((D, D)), _w_spec((1, D))],
        out_specs=[_row_spec(D), _row_spec(D)],
        out_shape=[jax.ShapeDtypeStruct((NP, D), jnp.float32),
                   jax.ShapeDtypeStruct((NP, D), jnp.float32)],
    )(acc2, rf1, root_w.reshape(1, D), ffn_w.reshape(1, D), w1, w3, w2,
      r2f_p["W"], r2f_p["b"].reshape(1, D))


@jax.jit
def _run(root_features, feedback_features, feedback_index, fringe_maps,
         root_to_fringe_index, root_edge_index, root_edge_attr, params):
    root_p = jnp.pad(root_features, ((0, NP - N), (0, 0)))
    ctx_p = jnp.pad(feedback_features, ((0, NP - N), (0, 0)))

    def pad_edges(ei):
        srcp = jnp.concatenate(
            [ei[0].astype(jnp.int32), jnp.zeros((E_PAD - E,), jnp.int32)])
        dstp = jnp.concatenate(
            [ei[1].astype(jnp.int32), jnp.full((E_PAD - E,), DUMMY, jnp.int32)])
        return srcp, dstp

    src1, dst1 = pad_edges(feedback_index)
    src2, dst2 = pad_edges(root_edge_index)
    attr_p = jnp.pad(root_edge_attr, ((0, E_PAD - E), (0, 0)))
    fidx_p = jnp.concatenate(
        [root_to_fringe_index.astype(jnp.int32),
         jnp.zeros((FP - F,), jnp.int32)])
    maps_p = jnp.pad(fringe_maps, ((0, FP - F), (0, 0)))

    hp = params["hgat"]
    btile = jnp.tile(hp["b_attn"], H).astype(jnp.float32)

    tsrc, adst = _tc_prep(root_p, ctx_p, hp)
    acc1 = _sc_hgat(tsrc, adst, src1, dst1, btile)
    rf1, qt, kvt = _tc_mid(acc1, root_p, adst, hp["b_attn"],
                           params["node_fb_norm_w"], params["mha"])
    acc2 = _sc_mha(qt, kvt, src2, dst2, attr_p)
    rf3, proj = _tc_tail(acc2, rf1, params["root_fb_norm_w"],
                         params["ffn_norm_w"], params["ffn"],
                         params["root_to_fringe"])
    fringe = _sc_fringe(proj, fidx_p, maps_p)
    return rf3[:N], fringe[:F]


def kernel(root_features, feedback_features, feedback_index, fringe_maps,
           root_to_fringe_index, root_edge_index, root_edge_attr, params):
    return _run(root_features, feedback_features, feedback_index, fringe_maps,
                root_to_fringe_index, root_edge_index, root_edge_attr, params)


# speedup vs baseline: 36.5051x; 1.8271x over previous
import functools
from math import sqrt

import jax
import jax.numpy as jnp
from jax import lax
from jax.experimental import pallas as pl
from jax.experimental.pallas import tpu as pltpu
from jax.experimental.pallas import tpu_sc as plsc

N = 10000
E = 320000
D = 128
F = 20000
H = 4
HD = 32
NP = 10240
DUMMY = 10016
R = 144
NC = 2
NS = 16
NW = NC * NS
EPW = 10240
E_PAD = NW * EPW
CHG = 40
NCHG = EPW // CHG
CHM = 32
NCHM = EPW // CHM
IB = 8
FP = 20480
FPW = FP // NW
CH = 128
BR = 2048
NB = NP // BR
HP = 384

_mesh = plsc.VectorSubcoreMesh(core_axis_name="c", subcore_axis_name="s",
                               num_cores=NC, num_subcores=NS)
_sc_params = pltpu.CompilerParams(use_tc_tiling_on_sc=False,
                                  needs_layout_passes=False)


def _zero_accum(s, zbuf, accum, sem, crows):
    rows0 = s * (NP // NS)
    ncp = (NP // NS) // crows

    @pl.loop(0, ncp)
    def _(i):
        pltpu.async_copy(zbuf, accum.at[pl.ds(rows0 + i * crows, crows)], sem)

    @pl.loop(0, ncp)
    def _(i):
        pltpu.make_async_copy(
            zbuf, accum.at[pl.ds(rows0 + i * crows, crows)], sem).wait()


def _zero_buf(buf, rows, cols):
    z16 = jnp.zeros((16,), jnp.float32)

    @pl.loop(0, rows)
    def _(r):
        for jj in range(cols // 16):
            buf[r, pl.ds(jj * 16, 16)] = z16


def _edge_pipeline(*, w, c_sz, nch, src2d, dst2d, isb, idb, gspecs, lspecs,
                   ubufs, accum, gsems, ssems, isem, compute):
    nblk = nch // IB
    row0 = w * nch
    base0 = w * nch * c_sz

    def issue(slot, bb, row, base, start):
        for tbl, bufs, which in gspecs:
            iref = (isb if which == "s" else idb)[bb].at[row]
            cp = pltpu.make_async_copy(tbl.at[iref], bufs[slot], gsems[slot])
            cp.start() if start else cp.wait()
        for tbl, bufs in lspecs:
            cp = pltpu.make_async_copy(tbl.at[pl.ds(base, c_sz)], bufs[slot],
                                       gsems[slot])
            cp.start() if start else cp.wait()

    def idx_load(kb1, bb, start):
        for hbm, bufs in ((src2d, isb), (dst2d, idb)):
            cp = pltpu.make_async_copy(hbm.at[pl.ds(row0 + kb1 * IB, IB)],
                                       bufs[bb], isem)
            cp.start() if start else cp.wait()

    pltpu.sync_copy(src2d.at[pl.ds(row0, IB)], isb[0])
    pltpu.sync_copy(dst2d.at[pl.ds(row0, IB)], idb[0])
    issue(0, 0, 0, base0, True)
    issue(1, 0, 1, base0 + c_sz, True)
    idx_load(1, 1, True)
    pltpu.async_copy(ubufs[0], accum.at[idb[0].at[0]], ssems[0], add=True)
    pltpu.async_copy(ubufs[1], accum.at[idb[0].at[1]], ssems[1], add=True)

    @pl.loop(0, nblk, step=2)
    def _(k):
        for kk in range(2):
            kb = k + kk
            for jb in range(IB):
                slot = jb % 2
                ch = kb * IB + jb
                issue(slot, kk, jb, base0 + ch * c_sz, False)
                pltpu.make_async_copy(ubufs[slot],
                                      accum.at[idb[kk].at[jb]],
                                      ssems[slot]).wait()
                compute(slot)
                pltpu.async_copy(ubufs[slot], accum.at[idb[kk].at[jb]],
                                 ssems[slot], add=True)
                if jb == 2:
                    @pl.when(jnp.logical_and(kb >= 1, kb + 1 < nblk))
                    def _():
                        idx_load(kb + 1, 1 - kk, True)
                if jb == IB - 2:
                    @pl.when(kb < nblk - 1)
                    def _():
                        idx_load(kb + 1, 1 - kk, False)
                if jb < IB - 2:
                    issue(slot, kk, jb + 2, base0 + (ch + 2) * c_sz, True)
                else:
                    @pl.when(kb < nblk - 1)
                    def _():
                        issue(slot, 1 - kk, jb + 2 - IB,
                              base0 + (ch + 2) * c_sz, True)

    for b in range(2):
        pltpu.make_async_copy(ubufs[b], accum.at[idb[0].at[b]],
                              ssems[b]).wait()


def _copy_out(c, s, accum, out):
    plsc.subcore_barrier()
    rows0 = s * (NP // NS)
    pltpu.sync_copy(accum.at[pl.ds(rows0, NP // NS)],
                    out.at[c, pl.ds(rows0, NP // NS)])


@functools.partial(
    pl.kernel,
    out_type=jax.ShapeDtypeStruct((NC, NP, R), jnp.float32),
    mesh=_mesh,
    compiler_params=_sc_params,
    scratch_types=[
        pltpu.VMEM((IB, CHG), jnp.int32), pltpu.VMEM((IB, CHG), jnp.int32),
        pltpu.VMEM((IB, CHG), jnp.int32), pltpu.VMEM((IB, CHG), jnp.int32),
        pltpu.VMEM((CHG, R), jnp.float32), pltpu.VMEM((CHG, R), jnp.float32),
        pltpu.VMEM((CHG, 16), jnp.float32), pltpu.VMEM((CHG, 16), jnp.float32),
        pltpu.VMEM((CHG, R), jnp.float32), pltpu.VMEM((CHG, R), jnp.float32),
        pltpu.VMEM((16,), jnp.float32),
        pltpu.VMEM_SHARED((NP, R), jnp.float32),
        pltpu.SemaphoreType.DMA, pltpu.SemaphoreType.DMA,
        pltpu.SemaphoreType.DMA, pltpu.SemaphoreType.DMA,
        pltpu.SemaphoreType.DMA,
    ],
)
def _sc_hgat(tsrc, adst, src2d, dst2d, btile, out,
             isb0, isb1, idb0, idb1, bs0, bs1, bd0, bd1, bu0, bu1,
             bvec, accum, gsem0, gsem1, ssem0, ssem1, isem):
    c = lax.axis_index("c")
    s = lax.axis_index("s")
    w = c * NS + s
    bufS, bufD, bufU = (bs0, bs1), (bd0, bd1), (bu0, bu1)

    pltpu.sync_copy(btile, bvec)
    _zero_buf(bu0, CHG, R)
    _zero_buf(bu1, CHG, R)
    _zero_accum(s, bu0, accum, isem, CHG)
    plsc.subcore_barrier()

    bv = bvec[...]
    lai = lax.iota(jnp.int32, 16)
    masks = [(lai == h).astype(jnp.float32) for h in range(H)]

    def compute(b):
        bs, bd, bu = bufS[b], bufD[b], bufU[b]

        @pl.loop(0, CHG)
        def _(e):
            aS = bs[e, pl.ds(128, 16)]
            aD = bd[e, pl.ds(0, 16)]
            z = aS + aD + bv
            l = jnp.where(z >= 0.0, z, z * 0.2)
            exvs = []
            for h in range(H):
                sh = jnp.sum(l * masks[h])
                exv = jnp.exp(lax.broadcast(sh, (16,)))
                exvs.append(exv)
                for q in range(2):
                    cc = h * 32 + q * 16
                    bu[e, pl.ds(cc, 16)] = bs[e, pl.ds(cc, 16)] * exv
            exrow = exvs[0]
            for h in range(1, H):
                exrow = jnp.where(lai == h, exvs[h], exrow)
            bu[e, pl.ds(128, 16)] = exrow

    _edge_pipeline(w=w, c_sz=CHG, nch=NCHG, src2d=src2d, dst2d=dst2d,
                   isb=(isb0, isb1), idb=(idb0, idb1),
                   gspecs=[(tsrc, bufS, "s"), (adst, bufD, "d")],
                   lspecs=[], ubufs=bufU, accum=accum,
                   gsems=(gsem0, gsem1), ssems=(ssem0, ssem1), isem=isem,
                   compute=compute)
    _copy_out(c, s, accum, out)


@functools.partial(
    pl.kernel,
    out_type=jax.ShapeDtypeStruct((NC, NP, R), jnp.float32),
    mesh=_mesh,
    compiler_params=_sc_params,
    scratch_types=[
        pltpu.VMEM((IB, CHM), jnp.int32), pltpu.VMEM((IB, CHM), jnp.int32),
        pltpu.VMEM((IB, CHM), jnp.int32), pltpu.VMEM((IB, CHM), jnp.int32),
        pltpu.VMEM((CHM, D), jnp.float32), pltpu.VMEM((CHM, D), jnp.float32),
        pltpu.VMEM((CHM, 2 * D), jnp.float32),
        pltpu.VMEM((CHM, 2 * D), jnp.float32),
        pltpu.VMEM((CHM, HD), jnp.float32), pltpu.VMEM((CHM, HD), jnp.float32),
        pltpu.VMEM((CHM, R), jnp.float32), pltpu.VMEM((CHM, R), jnp.float32),
        pltpu.VMEM_SHARED((NP, R), jnp.float32),
        pltpu.SemaphoreType.DMA, pltpu.SemaphoreType.DMA,
        pltpu.SemaphoreType.DMA, pltpu.SemaphoreType.DMA,
        pltpu.SemaphoreType.DMA,
    ],
)
def _sc_mha(qt, kvt, src2d, dst2d, attr, out,
            isb0, isb1, idb0, idb1, bq0, bq1, bkv0, bkv1, ba0, ba1, bu0, bu1,
            accum, gsem0, gsem1, ssem0, ssem1, isem):
    c = lax.axis_index("c")
    s = lax.axis_index("s")
    w = c * NS + s
    bufQ, bufKV, bufA, bufU = (bq0, bq1), (bkv0, bkv1), (ba0, ba1), (bu0, bu1)

    _zero_buf(bu0, CHM, R)
    _zero_buf(bu1, CHM, R)
    _zero_accum(s, bu0, accum, isem, CHM)
    plsc.subcore_barrier()

    lai = lax.iota(jnp.int32, 16)

    def compute(b):
        bq, bkv, ba, bu = bufQ[b], bufKV[b], bufA[b], bufU[b]

        @pl.loop(0, CHM)
        def _(e):
            a0 = ba[e, pl.ds(0, 16)]
            a1 = ba[e, pl.ds(16, 16)]
            exvs = []
            for h in range(H):
                q0 = bq[e, pl.ds(h * 32, 16)]
                q1 = bq[e, pl.ds(h * 32 + 16, 16)]
                k0 = bkv[e, pl.ds(h * 32, 16)]
                k1 = bkv[e, pl.ds(h * 32 + 16, 16)]
                p = q0 * (k0 * a0) + q1 * (k1 * a1)
                sh = jnp.sum(p)
                exv = jnp.exp(lax.broadcast(sh, (16,)))
                exvs.append(exv)
                for q in range(2):
                    cc = h * 32 + q * 16
                    bu[e, pl.ds(cc, 16)] = bkv[e, pl.ds(128 + cc, 16)] * exv
            exrow = exvs[0]
            for h in range(1, H):
                exrow = jnp.where(lai == h, exvs[h], exrow)
            bu[e, pl.ds(128, 16)] = exrow

    _edge_pipeline(w=w, c_sz=CHM, nch=NCHM, src2d=src2d, dst2d=dst2d,
                   isb=(isb0, isb1), idb=(idb0, idb1),
                   gspecs=[(qt, bufQ, "d"), (kvt, bufKV, "s")],
                   lspecs=[(attr, bufA)], ubufs=bufU, accum=accum,
                   gsems=(gsem0, gsem1), ssems=(ssem0, ssem1), isem=isem,
                   compute=compute)
    _copy_out(c, s, accum, out)


@functools.partial(
    pl.kernel,
    out_type=jax.ShapeDtypeStruct((FP, D), jnp.float32),
    mesh=_mesh,
    compiler_params=_sc_params,
    scratch_types=[
        pltpu.VMEM((CH,), jnp.int32),
        pltpu.VMEM((CH, D), jnp.float32),
        pltpu.VMEM((CH, D), jnp.float32),
        pltpu.SemaphoreType.DMA,
    ],
)
def _sc_fringe(proj, fidx, maps, out, idxb, bufP, bufM, sem):
    c = lax.axis_index("c")
    s = lax.axis_index("s")
    w = c * NS + s

    @pl.loop(0, FPW // CH)
    def _(ch):
        base = w * FPW + ch * CH
        pltpu.sync_copy(fidx.at[pl.ds(base, CH)], idxb)
        cp = pltpu.async_copy(proj.at[idxb], bufP, sem)
        pltpu.sync_copy(maps.at[pl.ds(base, CH)], bufM)
        cp.wait()

        @pl.loop(0, CH)
        def _(r):
            for jj in range(D // 16):
                bufP[r, pl.ds(jj * 16, 16)] = (
                    bufP[r, pl.ds(jj * 16, 16)] * bufM[r, pl.ds(jj * 16, 16)])

        pltpu.sync_copy(bufP, out.at[pl.ds(base, CH)])


def _mm(a, b):
    return jnp.dot(a, b, preferred_element_type=jnp.float32,
                   precision=lax.Precision.HIGHEST)


def _rms(x, w):
    return x * lax.rsqrt(jnp.mean(x * x, axis=-1, keepdims=True) + 1e-6) * w


def _bcast_heads(a4, nrows):
    return jnp.concatenate(
        [jnp.broadcast_to(a4[:, h:h + 1], (nrows, HD)) for h in range(H)],
        axis=1)


def _tc_prep_body(root, ctx, wc2x, bc2x, wx2c, bx2c, was, wad, tsrc, adst):
    cx = ctx[...]
    vals = _mm(cx, wc2x[...]) + bc2x[...]
    a_s = _mm(cx, was[...])
    tsrc[...] = jnp.concatenate([vals, a_s], axis=1)
    pre = _mm(root[...], wx2c[...]) + bx2c[...]
    adst[...] = _mm(pre, wad[...])


def _tc_mid_body(acc, root, adst, b16, wn, wq, bq, wk, bk, wv, bv,
                 rf1o, qo, kvo):
    a = acc[...]
    ac = a[0] + a[1]
    rt = root[...]
    zs = adst[...] + b16[...]
    ex16 = jnp.exp(jnp.where(zs >= 0.0, zs, zs * 0.2))
    ex4 = ex16[:, :H]
    numer = ac[:, :D] + _bcast_heads(ex4, BR) * rt
    denom = ac[:, D:D + H] + ex4
    hout = numer / (_bcast_heads(denom, BR) + 1e-16)
    rf1 = _rms(rt + hout, wn[...])
    rf1o[...] = rf1
    qo[...] = (_mm(rf1, wq[...]) + bq[...]) * (1.0 / sqrt(HD))
    kvo[...] = jnp.concatenate(
        [_mm(rf1, wk[...]) + bk[...], _mm(rf1, wv[...]) + bv[...]], axis=1)


def _tc_tail_body(acc, rf1, wn2, wn3, w1, w3, w2, wr2f, br2f, rf3o, projo):
    a = acc[...]
    ac = a[0] + a[1]
    numer = ac[:, :D]
    denom = ac[:, D:D + H]
    mo = numer / (_bcast_heads(denom, BR) + 1e-16)
    rf2 = _rms(rf1[...] + mo, wn2[...])
    h1 = _mm(rf2, w1[...])
    hs = h1 * jax.nn.sigmoid(h1) * _mm(rf2, w3[...])
    rf3 = _rms(rf2 + _mm(hs, w2[...]), wn3[...])
    rf3o[...] = rf3
    projo[...] = _mm(rf3, wr2f[...]) + br2f[...]


def _row_spec(cols):
    return pl.BlockSpec((BR, cols), lambda i: (i, 0))


def _w_spec(shape):
    nd = len(shape)
    return pl.BlockSpec(shape, lambda i: (0,) * nd)


def _acc_spec():
    return pl.BlockSpec((NC, BR, R), lambda i: (0, i, 0))


def _tc_prep(root_p, ctx_p, p):
    was = jnp.pad(p["W_attn"][:D], ((0, 0), (0, 16 - H)))
    wad = jnp.pad(p["W_attn"][D:], ((0, 0), (0, 16 - H)))
    return pl.pallas_call(
        _tc_prep_body,
        grid=(NB,),
        in_specs=[_row_spec(D), _row_spec(D), _w_spec((D, D)), _w_spec((1, D)),
                  _w_spec((D, D)), _w_spec((1, D)), _w_spec((D, 16)),
                  _w_spec((D, 16))],
        out_specs=[_row_spec(R), _row_spec(16)],
        out_shape=[jax.ShapeDtypeStruct((NP, R), jnp.float32),
                   jax.ShapeDtypeStruct((NP, 16), jnp.float32)],
    )(root_p, ctx_p, p["W_ctx_to_x"], p["b_ctx_to_x"].reshape(1, D),
      p["W_x_to_ctx"], p["b_x_to_ctx"].reshape(1, D), was, wad)


def _tc_mid(acc, root_p, adst, b_attn, node_w, mha_p):
    wqkv = mha_p["W_qkv"].reshape(D, H, HD, 3)
    bqkv = mha_p["b_qkv"].reshape(H, HD, 3)
    wq = wqkv[..., 0].reshape(D, D)
    wk = wqkv[..., 1].reshape(D, D)
    wv = wqkv[..., 2].reshape(D, D)
    bq = bqkv[..., 0].reshape(1, D)
    bk = bqkv[..., 1].reshape(1, D)
    bv = bqkv[..., 2].reshape(1, D)
    b16 = jnp.pad(b_attn, (0, 16 - H)).reshape(1, 16)
    return pl.pallas_call(
        _tc_mid_body,
        grid=(NB,),
        in_specs=[_acc_spec(), _row_spec(D), _row_spec(16), _w_spec((1, 16)),
                  _w_spec((1, D)), _w_spec((D, D)), _w_spec((1, D)),
                  _w_spec((D, D)), _w_spec((1, D)), _w_spec((D, D)),
                  _w_spec((1, D))],
        out_specs=[_row_spec(D), _row_spec(D), _row_spec(2 * D)],
        out_shape=[jax.ShapeDtypeStruct((NP, D), jnp.float32),
                   jax.ShapeDtypeStruct((NP, D), jnp.float32),
                   jax.ShapeDtypeStruct((NP, 2 * D), jnp.float32)],
    )(acc, root_p, adst, b16, node_w.reshape(1, D), wq, bq, wk, bk, wv, bv)


def _tc_tail(acc2, rf1, root_w, ffn_w, ffn_p, r2f_p):
    w1 = jnp.pad(ffn_p["W1"], ((0, 0), (0, HP - ffn_p["W1"].shape[1])))
    w3 = jnp.pad(ffn_p["W3"], ((0, 0), (0, HP - ffn_p["W3"].shape[1])))
    w2 = jnp.pad(ffn_p["W2"], ((0, HP - ffn_p["W2"].shape[0]), (0, 0)))
    return pl.pallas_call(
        _tc_tail_body,
        grid=(NB,),
        in_specs=[_acc_spec(), _row_spec(D), _w_spec((1, D)), _w_spec((1, D)),
                  _w_spec((D, HP)), _w_spec((D, HP)), _w_spec((HP, D)),
                  _w_spec((D, D)), _w_spec((1, D))],
        out_specs=[_row_spec(D), _row_spec(D)],
        out_shape=[jax.ShapeDtypeStruct((NP, D), jnp.float32),
                   jax.ShapeDtypeStruct((NP, D), jnp.float32)],
    )(acc2, rf1, root_w.reshape(1, D), ffn_w.reshape(1, D), w1, w3, w2,
      r2f_p["W"], r2f_p["b"].reshape(1, D))


@jax.jit
def _run(root_features, feedback_features, feedback_index, fringe_maps,
         root_to_fringe_index, root_edge_index, root_edge_attr, params):
    root_p = jnp.pad(root_features, ((0, NP - N), (0, 0)))
    ctx_p = jnp.pad(feedback_features, ((0, NP - N), (0, 0)))

    def pad_edges(ei, c_sz):
        srcp = jnp.concatenate(
            [ei[0].astype(jnp.int32), jnp.zeros((E_PAD - E,), jnp.int32)])
        dstp = jnp.concatenate(
            [ei[1].astype(jnp.int32), jnp.full((E_PAD - E,), DUMMY, jnp.int32)])
        return (srcp.reshape(E_PAD // c_sz, c_sz),
                dstp.reshape(E_PAD // c_sz, c_sz))

    src1, dst1 = pad_edges(feedback_index, CHG)
    src2, dst2 = pad_edges(root_edge_index, CHM)
    attr_p = jnp.pad(root_edge_attr, ((0, E_PAD - E), (0, 0)))
    fidx_p = jnp.concatenate(
        [root_to_fringe_index.astype(jnp.int32),
         jnp.zeros((FP - F,), jnp.int32)])
    maps_p = jnp.pad(fringe_maps, ((0, FP - F), (0, 0)))

    hp = params["hgat"]
    btile = jnp.tile(hp["b_attn"], H).astype(jnp.float32)

    tsrc, adst = _tc_prep(root_p, ctx_p, hp)
    acc1 = _sc_hgat(tsrc, adst, src1, dst1, btile)
    rf1, qt, kvt = _tc_mid(acc1, root_p, adst, hp["b_attn"],
                           params["node_fb_norm_w"], params["mha"])
    acc2 = _sc_mha(qt, kvt, src2, dst2, attr_p)
    rf3, proj = _tc_tail(acc2, rf1, params["root_fb_norm_w"],
                         params["ffn_norm_w"], params["ffn"],
                         params["root_to_fringe"])
    fringe = _sc_fringe(proj, fidx_p, maps_p)
    return rf3[:N], fringe[:F]


def kernel(root_features, feedback_features, feedback_index, fringe_maps,
           root_to_fringe_index, root_edge_index, root_edge_attr, params):
    return _run(root_features, feedback_features, feedback_index, fringe_maps,
                root_to_fringe_index, root_edge_index, root_edge_attr, params)


# speedup vs baseline: 48.2668x; 1.3222x over previous
import functools
from math import sqrt

import jax
import jax.numpy as jnp
from jax import lax
from jax.experimental import pallas as pl
from jax.experimental.pallas import tpu as pltpu
from jax.experimental.pallas import tpu_sc as plsc

N = 10000
E = 320000
D = 128
F = 20000
H = 4
HD = 32
NP = 10240
DUMMY = 10016
R = 144
NC = 2
NS = 16
NW = NC * NS
EPW = 10240
E_PAD = NW * EPW
CHG = 40
NCHG = EPW // CHG
CHM = 32
NCHM = EPW // CHM
IB = 8
FP = 20480
FPW = FP // NW
CH = 128
BR = 2048
NB = NP // BR
HP = 384

_mesh = plsc.VectorSubcoreMesh(core_axis_name="c", subcore_axis_name="s",
                               num_cores=NC, num_subcores=NS)
_sc_params = pltpu.CompilerParams(use_tc_tiling_on_sc=False,
                                  needs_layout_passes=False)


def _zero_accum(s, zbuf, accum, sem, crows):
    rows0 = s * (NP // NS)
    ncp = (NP // NS) // crows

    @pl.loop(0, ncp)
    def _(i):
        pltpu.async_copy(zbuf, accum.at[pl.ds(rows0 + i * crows, crows)], sem)

    @pl.loop(0, ncp)
    def _(i):
        pltpu.make_async_copy(
            zbuf, accum.at[pl.ds(rows0 + i * crows, crows)], sem).wait()


def _zero_buf(buf, rows, cols):
    z16 = jnp.zeros((16,), jnp.float32)

    @pl.loop(0, rows)
    def _(r):
        for jj in range(cols // 16):
            buf[r, pl.ds(jj * 16, 16)] = z16


def _edge_pipeline(*, w, c_sz, nch, src2d, dst2d, isb, idb, gspecs, lspecs,
                   ubufs, accum, gsems, ssems, isem, compute):
    nblk = nch // IB
    row0 = w * nch
    base0 = w * nch * c_sz

    def issue(slot, bb, row, base, start):
        for tbl, bufs, which in gspecs:
            iref = (isb if which == "s" else idb)[bb].at[row]
            cp = pltpu.make_async_copy(tbl.at[iref], bufs[slot], gsems[slot])
            cp.start() if start else cp.wait()
        for tbl, bufs in lspecs:
            cp = pltpu.make_async_copy(tbl.at[pl.ds(base, c_sz)], bufs[slot],
                                       gsems[slot])
            cp.start() if start else cp.wait()

    def idx_load(kb1, bb, start):
        for hbm, bufs in ((src2d, isb), (dst2d, idb)):
            cp = pltpu.make_async_copy(hbm.at[pl.ds(row0 + kb1 * IB, IB)],
                                       bufs[bb], isem)
            cp.start() if start else cp.wait()

    pltpu.sync_copy(src2d.at[pl.ds(row0, IB)], isb[0])
    pltpu.sync_copy(dst2d.at[pl.ds(row0, IB)], idb[0])
    issue(0, 0, 0, base0, True)
    issue(1, 0, 1, base0 + c_sz, True)
    idx_load(1, 1, True)
    pltpu.async_copy(ubufs[0], accum.at[idb[0].at[0]], ssems[0], add=True)
    pltpu.async_copy(ubufs[1], accum.at[idb[0].at[1]], ssems[1], add=True)

    @pl.loop(0, nblk, step=2)
    def _(k):
        for kk in range(2):
            kb = k + kk
            for jb in range(IB):
                slot = jb % 2
                ch = kb * IB + jb
                issue(slot, kk, jb, base0 + ch * c_sz, False)
                pltpu.make_async_copy(ubufs[slot],
                                      accum.at[idb[kk].at[jb]],
                                      ssems[slot]).wait()
                compute(slot)
                pltpu.async_copy(ubufs[slot], accum.at[idb[kk].at[jb]],
                                 ssems[slot], add=True)
                if jb == 2:
                    @pl.when(jnp.logical_and(kb >= 1, kb + 1 < nblk))
                    def _():
                        idx_load(kb + 1, 1 - kk, True)
                if jb == IB - 2:
                    @pl.when(kb < nblk - 1)
                    def _():
                        idx_load(kb + 1, 1 - kk, False)
                if jb < IB - 2:
                    issue(slot, kk, jb + 2, base0 + (ch + 2) * c_sz, True)
                else:
                    @pl.when(kb < nblk - 1)
                    def _():
                        issue(slot, 1 - kk, jb + 2 - IB,
                              base0 + (ch + 2) * c_sz, True)

    for b in range(2):
        pltpu.make_async_copy(ubufs[b], accum.at[idb[0].at[b]],
                              ssems[b]).wait()


def _copy_out(c, s, accum, out):
    plsc.subcore_barrier()
    rows0 = s * (NP // NS)
    pltpu.sync_copy(accum.at[pl.ds(rows0, NP // NS)],
                    out.at[c, pl.ds(rows0, NP // NS)])


@functools.partial(
    pl.kernel,
    out_type=jax.ShapeDtypeStruct((NC, NP, R), jnp.float32),
    mesh=_mesh,
    compiler_params=_sc_params,
    scratch_types=[
        pltpu.VMEM((IB, CHG), jnp.int32), pltpu.VMEM((IB, CHG), jnp.int32),
        pltpu.VMEM((IB, CHG), jnp.int32), pltpu.VMEM((IB, CHG), jnp.int32),
        pltpu.VMEM((CHG, R), jnp.float32), pltpu.VMEM((CHG, R), jnp.float32),
        pltpu.VMEM((CHG, 16), jnp.float32), pltpu.VMEM((CHG, 16), jnp.float32),
        pltpu.VMEM((CHG, R), jnp.float32), pltpu.VMEM((CHG, R), jnp.float32),
        pltpu.VMEM((16,), jnp.float32),
        pltpu.VMEM_SHARED((NP, R), jnp.float32),
        pltpu.SemaphoreType.DMA, pltpu.SemaphoreType.DMA,
        pltpu.SemaphoreType.DMA, pltpu.SemaphoreType.DMA,
        pltpu.SemaphoreType.DMA,
    ],
)
def _sc_hgat(tsrc, adst, src2d, dst2d, btile, out,
             isb0, isb1, idb0, idb1, bs0, bs1, bd0, bd1, bu0, bu1,
             bvec, accum, gsem0, gsem1, ssem0, ssem1, isem):
    c = lax.axis_index("c")
    s = lax.axis_index("s")
    w = c * NS + s
    bufS, bufD, bufU = (bs0, bs1), (bd0, bd1), (bu0, bu1)

    pltpu.sync_copy(btile, bvec)
    _zero_buf(bu0, CHG, R)
    _zero_buf(bu1, CHG, R)
    _zero_accum(s, bu0, accum, isem, CHG)
    plsc.subcore_barrier()

    bv = bvec[...]
    lai = lax.iota(jnp.int32, 16)
    masks = [(lai == h).astype(jnp.float32) for h in range(H)]

    def compute(b):
        bs, bd, bu = bufS[b], bufD[b], bufU[b]

        @plsc.parallel_loop(0, CHG, unroll=2)
        def _(e):
            aS = bs[e, pl.ds(128, 16)]
            aD = bd[e, pl.ds(0, 16)]
            z = aS + aD + bv
            l = jnp.where(z >= 0.0, z, z * 0.2)
            exvs = []
            for h in range(H):
                sh = jnp.sum(l * masks[h])
                exv = jnp.exp(lax.broadcast(sh, (16,)))
                exvs.append(exv)
                for q in range(2):
                    cc = h * 32 + q * 16
                    bu[e, pl.ds(cc, 16)] = bs[e, pl.ds(cc, 16)] * exv
            exrow = exvs[0]
            for h in range(1, H):
                exrow = jnp.where(lai == h, exvs[h], exrow)
            bu[e, pl.ds(128, 16)] = exrow

    _edge_pipeline(w=w, c_sz=CHG, nch=NCHG, src2d=src2d, dst2d=dst2d,
                   isb=(isb0, isb1), idb=(idb0, idb1),
                   gspecs=[(tsrc, bufS, "s"), (adst, bufD, "d")],
                   lspecs=[], ubufs=bufU, accum=accum,
                   gsems=(gsem0, gsem1), ssems=(ssem0, ssem1), isem=isem,
                   compute=compute)
    _copy_out(c, s, accum, out)


@functools.partial(
    pl.kernel,
    out_type=jax.ShapeDtypeStruct((NC, NP, R), jnp.float32),
    mesh=_mesh,
    compiler_params=_sc_params,
    scratch_types=[
        pltpu.VMEM((IB, CHM), jnp.int32), pltpu.VMEM((IB, CHM), jnp.int32),
        pltpu.VMEM((IB, CHM), jnp.int32), pltpu.VMEM((IB, CHM), jnp.int32),
        pltpu.VMEM((CHM, D), jnp.float32), pltpu.VMEM((CHM, D), jnp.float32),
        pltpu.VMEM((CHM, 2 * D), jnp.float32),
        pltpu.VMEM((CHM, 2 * D), jnp.float32),
        pltpu.VMEM((CHM, HD), jnp.float32), pltpu.VMEM((CHM, HD), jnp.float32),
        pltpu.VMEM((CHM, R), jnp.float32), pltpu.VMEM((CHM, R), jnp.float32),
        pltpu.VMEM_SHARED((NP, R), jnp.float32),
        pltpu.SemaphoreType.DMA, pltpu.SemaphoreType.DMA,
        pltpu.SemaphoreType.DMA, pltpu.SemaphoreType.DMA,
        pltpu.SemaphoreType.DMA,
    ],
)
def _sc_mha(qt, kvt, src2d, dst2d, attr, out,
            isb0, isb1, idb0, idb1, bq0, bq1, bkv0, bkv1, ba0, ba1, bu0, bu1,
            accum, gsem0, gsem1, ssem0, ssem1, isem):
    c = lax.axis_index("c")
    s = lax.axis_index("s")
    w = c * NS + s
    bufQ, bufKV, bufA, bufU = (bq0, bq1), (bkv0, bkv1), (ba0, ba1), (bu0, bu1)

    _zero_buf(bu0, CHM, R)
    _zero_buf(bu1, CHM, R)
    _zero_accum(s, bu0, accum, isem, CHM)
    plsc.subcore_barrier()

    lai = lax.iota(jnp.int32, 16)

    def compute(b):
        bq, bkv, ba, bu = bufQ[b], bufKV[b], bufA[b], bufU[b]

        @plsc.parallel_loop(0, CHM, unroll=2)
        def _(e):
            a0 = ba[e, pl.ds(0, 16)]
            a1 = ba[e, pl.ds(16, 16)]
            exvs = []
            for h in range(H):
                q0 = bq[e, pl.ds(h * 32, 16)]
                q1 = bq[e, pl.ds(h * 32 + 16, 16)]
                k0 = bkv[e, pl.ds(h * 32, 16)]
                k1 = bkv[e, pl.ds(h * 32 + 16, 16)]
                p = q0 * (k0 * a0) + q1 * (k1 * a1)
                sh = jnp.sum(p)
                exv = jnp.exp(lax.broadcast(sh, (16,)))
                exvs.append(exv)
                for q in range(2):
                    cc = h * 32 + q * 16
                    bu[e, pl.ds(cc, 16)] = bkv[e, pl.ds(128 + cc, 16)] * exv
            exrow = exvs[0]
            for h in range(1, H):
                exrow = jnp.where(lai == h, exvs[h], exrow)
            bu[e, pl.ds(128, 16)] = exrow

    _edge_pipeline(w=w, c_sz=CHM, nch=NCHM, src2d=src2d, dst2d=dst2d,
                   isb=(isb0, isb1), idb=(idb0, idb1),
                   gspecs=[(qt, bufQ, "d"), (kvt, bufKV, "s")],
                   lspecs=[(attr, bufA)], ubufs=bufU, accum=accum,
                   gsems=(gsem0, gsem1), ssems=(ssem0, ssem1), isem=isem,
                   compute=compute)
    _copy_out(c, s, accum, out)


@functools.partial(
    pl.kernel,
    out_type=jax.ShapeDtypeStruct((FP, D), jnp.float32),
    mesh=_mesh,
    compiler_params=_sc_params,
    scratch_types=[
        pltpu.VMEM((CH,), jnp.int32),
        pltpu.VMEM((CH, D), jnp.float32),
        pltpu.VMEM((CH, D), jnp.float32),
        pltpu.SemaphoreType.DMA,
    ],
)
def _sc_fringe(proj, fidx, maps, out, idxb, bufP, bufM, sem):
    c = lax.axis_index("c")
    s = lax.axis_index("s")
    w = c * NS + s

    @pl.loop(0, FPW // CH)
    def _(ch):
        base = w * FPW + ch * CH
        pltpu.sync_copy(fidx.at[pl.ds(base, CH)], idxb)
        cp = pltpu.async_copy(proj.at[idxb], bufP, sem)
        pltpu.sync_copy(maps.at[pl.ds(base, CH)], bufM)
        cp.wait()

        @plsc.parallel_loop(0, CH, unroll=2)
        def _(r):
            for jj in range(D // 16):
                bufP[r, pl.ds(jj * 16, 16)] = (
                    bufP[r, pl.ds(jj * 16, 16)] * bufM[r, pl.ds(jj * 16, 16)])

        pltpu.sync_copy(bufP, out.at[pl.ds(base, CH)])


def _mm(a, b):
    return jnp.dot(a, b, preferred_element_type=jnp.float32,
                   precision=lax.Precision.HIGHEST)


def _rms(x, w):
    return x * lax.rsqrt(jnp.mean(x * x, axis=-1, keepdims=True) + 1e-6) * w


def _bcast_heads(a4, nrows):
    return jnp.concatenate(
        [jnp.broadcast_to(a4[:, h:h + 1], (nrows, HD)) for h in range(H)],
        axis=1)


def _tc_prep_body(root, ctx, wc2x, bc2x, wx2c, bx2c, was, wad, tsrc, adst):
    cx = ctx[...]
    vals = _mm(cx, wc2x[...]) + bc2x[...]
    a_s = _mm(cx, was[...])
    tsrc[...] = jnp.concatenate([vals, a_s], axis=1)
    pre = _mm(root[...], wx2c[...]) + bx2c[...]
    adst[...] = _mm(pre, wad[...])


def _tc_mid_body(acc, root, adst, b16, wn, wq, bq, wk, bk, wv, bv,
                 rf1o, qo, kvo):
    a = acc[...]
    ac = a[0] + a[1]
    rt = root[...]
    zs = adst[...] + b16[...]
    ex16 = jnp.exp(jnp.where(zs >= 0.0, zs, zs * 0.2))
    ex4 = ex16[:, :H]
    numer = ac[:, :D] + _bcast_heads(ex4, BR) * rt
    denom = ac[:, D:D + H] + ex4
    hout = numer / (_bcast_heads(denom, BR) + 1e-16)
    rf1 = _rms(rt + hout, wn[...])
    rf1o[...] = rf1
    qo[...] = (_mm(rf1, wq[...]) + bq[...]) * (1.0 / sqrt(HD))
    kvo[...] = jnp.concatenate(
        [_mm(rf1, wk[...]) + bk[...], _mm(rf1, wv[...]) + bv[...]], axis=1)


def _tc_tail_body(acc, rf1, wn2, wn3, w1, w3, w2, wr2f, br2f, rf3o, projo):
    a = acc[...]
    ac = a[0] + a[1]
    numer = ac[:, :D]
    denom = ac[:, D:D + H]
    mo = numer / (_bcast_heads(denom, BR) + 1e-16)
    rf2 = _rms(rf1[...] + mo, wn2[...])
    h1 = _mm(rf2, w1[...])
    hs = h1 * jax.nn.sigmoid(h1) * _mm(rf2, w3[...])
    rf3 = _rms(rf2 + _mm(hs, w2[...]), wn3[...])
    rf3o[...] = rf3
    projo[...] = _mm(rf3, wr2f[...]) + br2f[...]


def _row_spec(cols):
    return pl.BlockSpec((BR, cols), lambda i: (i, 0))


def _w_spec(shape):
    nd = len(shape)
    return pl.BlockSpec(shape, lambda i: (0,) * nd)


def _acc_spec():
    return pl.BlockSpec((NC, BR, R), lambda i: (0, i, 0))


def _tc_prep(root_p, ctx_p, p):
    was = jnp.pad(p["W_attn"][:D], ((0, 0), (0, 16 - H)))
    wad = jnp.pad(p["W_attn"][D:], ((0, 0), (0, 16 - H)))
    return pl.pallas_call(
        _tc_prep_body,
        grid=(NB,),
        in_specs=[_row_spec(D), _row_spec(D), _w_spec((D, D)), _w_spec((1, D)),
                  _w_spec((D, D)), _w_spec((1, D)), _w_spec((D, 16)),
                  _w_spec((D, 16))],
        out_specs=[_row_spec(R), _row_spec(16)],
        out_shape=[jax.ShapeDtypeStruct((NP, R), jnp.float32),
                   jax.ShapeDtypeStruct((NP, 16), jnp.float32)],
    )(root_p, ctx_p, p["W_ctx_to_x"], p["b_ctx_to_x"].reshape(1, D),
      p["W_x_to_ctx"], p["b_x_to_ctx"].reshape(1, D), was, wad)


def _tc_mid(acc, root_p, adst, b_attn, node_w, mha_p):
    wqkv = mha_p["W_qkv"].reshape(D, H, HD, 3)
    bqkv = mha_p["b_qkv"].reshape(H, HD, 3)
    wq = wqkv[..., 0].reshape(D, D)
    wk = wqkv[..., 1].reshape(D, D)
    wv = wqkv[..., 2].reshape(D, D)
    bq = bqkv[..., 0].reshape(1, D)
    bk = bqkv[..., 1].reshape(1, D)
    bv = bqkv[..., 2].reshape(1, D)
    b16 = jnp.pad(b_attn, (0, 16 - H)).reshape(1, 16)
    return pl.pallas_call(
        _tc_mid_body,
        grid=(NB,),
        in_specs=[_acc_spec(), _row_spec(D), _row_spec(16), _w_spec((1, 16)),
                  _w_spec((1, D)), _w_spec((D, D)), _w_spec((1, D)),
                  _w_spec((D, D)), _w_spec((1, D)), _w_spec((D, D)),
                  _w_spec((1, D))],
        out_specs=[_row_spec(D), _row_spec(D), _row_spec(2 * D)],
        out_shape=[jax.ShapeDtypeStruct((NP, D), jnp.float32),
                   jax.ShapeDtypeStruct((NP, D), jnp.float32),
                   jax.ShapeDtypeStruct((NP, 2 * D), jnp.float32)],
    )(acc, root_p, adst, b16, node_w.reshape(1, D), wq, bq, wk, bk, wv, bv)


def _tc_tail(acc2, rf1, root_w, ffn_w, ffn_p, r2f_p):
    w1 = jnp.pad(ffn_p["W1"], ((0, 0), (0, HP - ffn_p["W1"].shape[1])))
    w3 = jnp.pad(ffn_p["W3"], ((0, 0), (0, HP - ffn_p["W3"].shape[1])))
    w2 = jnp.pad(ffn_p["W2"], ((0, HP - ffn_p["W2"].shape[0]), (0, 0)))
    return pl.pallas_call(
        _tc_tail_body,
        grid=(NB,),
        in_specs=[_acc_spec(), _row_spec(D), _w_spec((1, D)), _w_spec((1, D)),
                  _w_spec((D, HP)), _w_spec((D, HP)), _w_spec((HP, D)),
                  _w_spec((D, D)), _w_spec((1, D))],
        out_specs=[_row_spec(D), _row_spec(D)],
        out_shape=[jax.ShapeDtypeStruct((NP, D), jnp.float32),
                   jax.ShapeDtypeStruct((NP, D), jnp.float32)],
    )(acc2, rf1, root_w.reshape(1, D), ffn_w.reshape(1, D), w1, w3, w2,
      r2f_p["W"], r2f_p["b"].reshape(1, D))


@jax.jit
def _run(root_features, feedback_features, feedback_index, fringe_maps,
         root_to_fringe_index, root_edge_index, root_edge_attr, params):
    root_p = jnp.pad(root_features, ((0, NP - N), (0, 0)))
    ctx_p = jnp.pad(feedback_features, ((0, NP - N), (0, 0)))

    def pad_edges(ei, c_sz):
        srcp = jnp.concatenate(
            [ei[0].astype(jnp.int32), jnp.zeros((E_PAD - E,), jnp.int32)])
        dstp = jnp.concatenate(
            [ei[1].astype(jnp.int32), jnp.full((E_PAD - E,), DUMMY, jnp.int32)])
        return (srcp.reshape(E_PAD // c_sz, c_sz),
                dstp.reshape(E_PAD // c_sz, c_sz))

    src1, dst1 = pad_edges(feedback_index, CHG)
    src2, dst2 = pad_edges(root_edge_index, CHM)
    attr_p = jnp.pad(root_edge_attr, ((0, E_PAD - E), (0, 0)))
    fidx_p = jnp.concatenate(
        [root_to_fringe_index.astype(jnp.int32),
         jnp.zeros((FP - F,), jnp.int32)])
    maps_p = jnp.pad(fringe_maps, ((0, FP - F), (0, 0)))

    hp = params["hgat"]
    btile = jnp.tile(hp["b_attn"], H).astype(jnp.float32)

    tsrc, adst = _tc_prep(root_p, ctx_p, hp)
    acc1 = _sc_hgat(tsrc, adst, src1, dst1, btile)
    rf1, qt, kvt = _tc_mid(acc1, root_p, adst, hp["b_attn"],
                           params["node_fb_norm_w"], params["mha"])
    acc2 = _sc_mha(qt, kvt, src2, dst2, attr_p)
    rf3, proj = _tc_tail(acc2, rf1, params["root_fb_norm_w"],
                         params["ffn_norm_w"], params["ffn"],
                         params["root_to_fringe"])
    fringe = _sc_fringe(proj, fidx_p, maps_p)
    return rf3[:N], fringe[:F]


def kernel(root_features, feedback_features, feedback_index, fringe_maps,
           root_to_fringe_index, root_edge_index, root_edge_attr, params):
    return _run(root_features, feedback_features, feedback_index, fringe_maps,
                root_to_fringe_index, root_edge_index, root_edge_attr, params)
